# Initial kernel scaffold; baseline (speedup 1.0000x reference)
import jax
import jax.numpy as jnp
from jax.experimental import pallas as pl


def kernel(x, edge_index, edge_attr, W1n, b1n, W1e, b1e, W2n, b2n, W2e, b2e, Wg, bg, Wo, bo):
    raise NotImplementedError("write your pallas kernel here")


# speedup vs baseline: 6.7036x; 6.7036x over previous
import dataclasses
import functools

import jax
import jax.numpy as jnp
from jax import lax
from jax.experimental import pallas as pl
from jax.experimental.pallas import tpu as pltpu
from jax.experimental.pallas import tpu_sc as plsc

N = 10000
E = 320000
F = 128
DE = 16
C = 40

NCORE = 2
NSUB = 16
NW = NCORE * NSUB
CH = 128
NCHUNK = E // CH
ITERS = (NCHUNK + NW - 1) // NW
NPAD = 10112
RPS = NPAD // NSUB
ZR = 8

_mesh = plsc.VectorSubcoreMesh(core_axis_name="c", subcore_axis_name="s")

_sc_params = pltpu.CompilerParams()
if "needs_layout_passes" in pltpu.CompilerParams.__dataclass_fields__:
  _sc_params = dataclasses.replace(_sc_params, needs_layout_passes=False)


def _zero_accum(z_buf, accum, base0):
  @pl.loop(0, RPS // ZR)
  def _(k):
    pltpu.sync_copy(z_buf, accum.at[pl.ds(base0 + k * ZR, ZR)])


def _make_gine_pass(with_deg):
  agg_t = jax.ShapeDtypeStruct((NCORE, NPAD, F), jnp.float32)
  if with_deg:
    out_type = (agg_t, jax.ShapeDtypeStruct((NW, NPAD), jnp.float32))
  else:
    out_type = agg_t
  scratch = [
      pltpu.VMEM((2, CH), jnp.int32),
      pltpu.VMEM((CH, F), jnp.float32),
      pltpu.VMEM((CH, F), jnp.float32),
      pltpu.VMEM((ZR, F), jnp.float32),
      pltpu.VMEM((NPAD,), jnp.float32),
      pltpu.VMEM_SHARED((NPAD, F), jnp.float32),
      pltpu.SemaphoreType.DMA,
      pltpu.SemaphoreType.DMA,
  ]

  @functools.partial(pl.kernel, out_type=out_type, mesh=_mesh,
                     scratch_types=scratch, compiler_params=_sc_params)
  def gine_pass(h_hbm, e_hbm, ei_hbm, *refs):
    if with_deg:
      agg_hbm, deg_hbm = refs[0], refs[1]
      refs = refs[2:]
    else:
      agg_hbm = refs[0]
      refs = refs[1:]
    idx_v, g_buf, e_buf, z_buf, deg_t, accum, sem_g, sem_e = refs

    cid = lax.axis_index("c")
    sid = lax.axis_index("s")
    wid = cid * NSUB + sid
    base0 = sid * RPS

    @pl.loop(0, ZR)
    def _(r):
      for c in range(F // 16):
        z_buf[r, pl.ds(c * 16, 16)] = jnp.zeros((16,), jnp.float32)

    if with_deg:
      @pl.loop(0, NPAD // 16)
      def _(i):
        deg_t[pl.ds(i * 16, 16)] = jnp.zeros((16,), jnp.float32)

    _zero_accum(z_buf, accum, base0)
    plsc.subcore_barrier()

    @pl.loop(0, ITERS)
    def _(it):
      chunk = it * NW + wid

      @pl.when(chunk < NCHUNK)
      def _():
        base = pl.multiple_of(chunk * CH, CH)
        pltpu.sync_copy(ei_hbm.at[:, pl.ds(base, CH)], idx_v)
        cp_g = pltpu.async_copy(h_hbm.at[idx_v.at[0]], g_buf, sem_g)
        cp_e = pltpu.async_copy(e_hbm.at[chunk], e_buf, sem_e)
        cp_g.wait()
        cp_e.wait()

        @pl.loop(0, CH)
        def _(r):
          for c in range(F // 16):
            sl = (r, pl.ds(c * 16, 16))
            g_buf[sl] = jnp.maximum(g_buf[sl] + e_buf[sl], 0.0)

        pltpu.sync_copy(g_buf, accum.at[idx_v.at[1]], add=True)
        if with_deg:
          ones16 = jnp.ones((16,), jnp.float32)
          for c in range(CH // 16):
            d16 = idx_v[1, pl.ds(c * 16, 16)]
            plsc.addupdate_scatter(deg_t, [d16], ones16)

    plsc.subcore_barrier()
    pltpu.sync_copy(accum.at[pl.ds(base0, RPS)],
                    agg_hbm.at[cid, pl.ds(base0, RPS)])

    if with_deg:
      pltpu.sync_copy(deg_t, deg_hbm.at[wid])

  return gine_pass


_gine_pass_deg = _make_gine_pass(True)
_gine_pass = _make_gine_pass(False)


@functools.partial(
    pl.kernel,
    out_type=jax.ShapeDtypeStruct((NCORE, NPAD, F), jnp.float32),
    mesh=_mesh,
    compiler_params=_sc_params,
    scratch_types=[
        pltpu.VMEM((2, CH), jnp.int32),
        pltpu.VMEM((CH, F), jnp.float32),
        pltpu.VMEM((ZR, F), jnp.float32),
        pltpu.VMEM_SHARED((NPAD, F), jnp.float32),
        pltpu.SemaphoreType.DMA,
    ])
def _gcn_pass(y_hbm, ei_hbm, agg_hbm, idx_v, g_buf, z_buf, accum, sem_g):
  cid = lax.axis_index("c")
  sid = lax.axis_index("s")
  wid = cid * NSUB + sid
  base0 = sid * RPS

  @pl.loop(0, ZR)
  def _(r):
    for c in range(F // 16):
      z_buf[r, pl.ds(c * 16, 16)] = jnp.zeros((16,), jnp.float32)

  _zero_accum(z_buf, accum, base0)
  plsc.subcore_barrier()

  @pl.loop(0, ITERS)
  def _(it):
    chunk = it * NW + wid

    @pl.when(chunk < NCHUNK)
    def _():
      base = pl.multiple_of(chunk * CH, CH)
      pltpu.sync_copy(ei_hbm.at[:, pl.ds(base, CH)], idx_v)
      pltpu.async_copy(y_hbm.at[idx_v.at[0]], g_buf, sem_g).wait()
      pltpu.sync_copy(g_buf, accum.at[idx_v.at[1]], add=True)

  plsc.subcore_barrier()
  pltpu.sync_copy(accum.at[pl.ds(base0, RPS)],
                  agg_hbm.at[cid, pl.ds(base0, RPS)])


_BE = 4000
_BN = 2048


def _edge_mm_body(a_ref, w_ref, b_ref, o_ref):
  o_ref[...] = jnp.dot(a_ref[...], w_ref[...],
                       precision=lax.Precision.HIGHEST,
                       preferred_element_type=jnp.float32) + b_ref[...]


def _edge_mm(edge_attr, We, be):
  return pl.pallas_call(
      _edge_mm_body,
      grid=(E // _BE,),
      in_specs=[
          pl.BlockSpec((_BE, DE), lambda i: (i, 0)),
          pl.BlockSpec((DE, F), lambda i: (0, 0)),
          pl.BlockSpec((1, F), lambda i: (0, 0)),
      ],
      out_specs=pl.BlockSpec((_BE, F), lambda i: (i, 0)),
      out_shape=jax.ShapeDtypeStruct((E, F), jnp.float32),
  )(edge_attr, We, be.reshape(1, F))


def _sum_p(p_ref):
  return p_ref[0] + p_ref[1]


def _gine_node_body(x_ref, p_ref, w_ref, b_ref, o_ref):
  a = x_ref[...] + _sum_p(p_ref)
  h = jnp.dot(a, w_ref[...], precision=lax.Precision.HIGHEST,
              preferred_element_type=jnp.float32) + b_ref[...]
  o_ref[...] = jnp.maximum(h, 0.0)


def _gine_node(x, p, Wn, bn):
  return pl.pallas_call(
      _gine_node_body,
      grid=(pl.cdiv(N, _BN),),
      in_specs=[
          pl.BlockSpec((_BN, F), lambda i: (i, 0)),
          pl.BlockSpec((NCORE, _BN, F), lambda i: (0, i, 0)),
          pl.BlockSpec((F, F), lambda i: (0, 0)),
          pl.BlockSpec((1, F), lambda i: (0, 0)),
      ],
      out_specs=pl.BlockSpec((_BN, F), lambda i: (i, 0)),
      out_shape=jax.ShapeDtypeStruct((N, F), jnp.float32),
  )(x, p, Wn, bn.reshape(1, F))


def _gcn_prep_body(h_ref, p_ref, d_ref, wn_ref, bn_ref, wg_ref,
                   xw_ref, y_ref, dinv_ref):
  a = h_ref[...] + _sum_p(p_ref)
  h2 = jnp.maximum(
      jnp.dot(a, wn_ref[...], precision=lax.Precision.HIGHEST,
              preferred_element_type=jnp.float32) + bn_ref[...], 0.0)
  deg = jnp.sum(d_ref[...], axis=0) + 1.0
  dinv = lax.rsqrt(deg)[:, None]
  xw = jnp.dot(h2, wg_ref[...], precision=lax.Precision.HIGHEST,
               preferred_element_type=jnp.float32)
  xw_ref[...] = xw
  y_ref[...] = xw * dinv
  dinv_ref[...] = dinv


def _gcn_prep(h1, p, degp, W2n, b2n, Wg):
  return pl.pallas_call(
      _gcn_prep_body,
      grid=(pl.cdiv(N, _BN),),
      in_specs=[
          pl.BlockSpec((_BN, F), lambda i: (i, 0)),
          pl.BlockSpec((NCORE, _BN, F), lambda i: (0, i, 0)),
          pl.BlockSpec((NW, _BN), lambda i: (0, i)),
          pl.BlockSpec((F, F), lambda i: (0, 0)),
          pl.BlockSpec((1, F), lambda i: (0, 0)),
          pl.BlockSpec((F, F), lambda i: (0, 0)),
      ],
      out_specs=[
          pl.BlockSpec((_BN, F), lambda i: (i, 0)),
          pl.BlockSpec((_BN, F), lambda i: (i, 0)),
          pl.BlockSpec((_BN, 1), lambda i: (i, 0)),
      ],
      out_shape=[
          jax.ShapeDtypeStruct((N, F), jnp.float32),
          jax.ShapeDtypeStruct((N, F), jnp.float32),
          jax.ShapeDtypeStruct((N, 1), jnp.float32),
      ],
  )(h1, p, degp, W2n, b2n.reshape(1, F), Wg)


def _final_body(p_ref, xw_ref, dinv_ref, bg_ref, wo_ref, bo_ref, o_ref):
  dv = dinv_ref[...]
  h3 = jnp.maximum(dv * _sum_p(p_ref) + dv * dv * xw_ref[...]
                   + bg_ref[...], 0.0)
  logits = jnp.dot(h3, wo_ref[...], precision=lax.Precision.HIGHEST,
                   preferred_element_type=jnp.float32) + bo_ref[...]
  m = jnp.max(logits, axis=1, keepdims=True)
  lse = m + jnp.log(jnp.sum(jnp.exp(logits - m), axis=1, keepdims=True))
  o_ref[...] = (logits - lse)[:, :C]


def _final(p, xw, dinv, bg, Wo_pad, bo_pad):
  return pl.pallas_call(
      _final_body,
      grid=(pl.cdiv(N, _BN),),
      in_specs=[
          pl.BlockSpec((NCORE, _BN, F), lambda i: (0, i, 0)),
          pl.BlockSpec((_BN, F), lambda i: (i, 0)),
          pl.BlockSpec((_BN, 1), lambda i: (i, 0)),
          pl.BlockSpec((1, F), lambda i: (0, 0)),
          pl.BlockSpec((F, F), lambda i: (0, 0)),
          pl.BlockSpec((1, F), lambda i: (0, 0)),
      ],
      out_specs=pl.BlockSpec((_BN, C), lambda i: (i, 0)),
      out_shape=jax.ShapeDtypeStruct((N, C), jnp.float32),
  )(p, xw, dinv, bg.reshape(1, F), Wo_pad, bo_pad.reshape(1, F))


def kernel(x, edge_index, edge_attr, W1n, b1n, W1e, b1e, W2n, b2n, W2e, b2e,
           Wg, bg, Wo, bo):
  e1 = _edge_mm(edge_attr, W1e, b1e).reshape(NCHUNK, CH, F)
  e2 = _edge_mm(edge_attr, W2e, b2e).reshape(NCHUNK, CH, F)

  p1, degp = _gine_pass_deg(x, e1, edge_index)
  h1 = _gine_node(x, p1, W1n, b1n)

  p2 = _gine_pass(h1, e2, edge_index)
  xw, y, dinv = _gcn_prep(h1, p2, degp, W2n, b2n, Wg)

  p3 = _gcn_pass(y, edge_index)

  Wo_pad = jnp.pad(Wo, ((0, 0), (0, F - C)))
  bo_pad = jnp.pad(bo, (0, F - C), constant_values=-1e30)
  return _final(p3, xw, dinv, bg, Wo_pad, bo_pad)


# speedup vs baseline: 7.9792x; 1.1903x over previous
import dataclasses
import functools

import jax
import jax.numpy as jnp
from jax import lax
from jax.experimental import pallas as pl
from jax.experimental.pallas import tpu as pltpu
from jax.experimental.pallas import tpu_sc as plsc

N = 10000
E = 320000
F = 128
DE = 16
C = 40

NCORE = 2
NSUB = 16
NW = NCORE * NSUB
CH = 128
NCHUNK = E // CH
ITERS = (NCHUNK + NW - 1) // NW
NPAD = 10112
RPS = NPAD // NSUB
ZR = 8

_mesh = plsc.VectorSubcoreMesh(core_axis_name="c", subcore_axis_name="s")

_sc_params = pltpu.CompilerParams()
if "needs_layout_passes" in pltpu.CompilerParams.__dataclass_fields__:
  _sc_params = dataclasses.replace(_sc_params, needs_layout_passes=False)


def _zero_accum(z_buf, accum, base0):
  @pl.loop(0, RPS // ZR)
  def _(k):
    pltpu.sync_copy(z_buf, accum.at[pl.ds(base0 + k * ZR, ZR)])


def _make_gine_pass(with_deg):
  agg_t = jax.ShapeDtypeStruct((NCORE, NPAD, F), jnp.float32)
  if with_deg:
    out_type = (agg_t, jax.ShapeDtypeStruct((NW, NPAD), jnp.float32))
  else:
    out_type = agg_t
  scratch = [
      pltpu.VMEM((2, CH), jnp.int32),
      pltpu.VMEM((CH, F), jnp.float32),
      pltpu.VMEM((CH, F), jnp.float32),
      pltpu.VMEM((ZR, F), jnp.float32),
      pltpu.VMEM((NPAD,), jnp.float32),
      pltpu.VMEM_SHARED((NPAD, F), jnp.float32),
      pltpu.SemaphoreType.DMA,
      pltpu.SemaphoreType.DMA,
  ]

  @functools.partial(pl.kernel, out_type=out_type, mesh=_mesh,
                     scratch_types=scratch, compiler_params=_sc_params)
  def gine_pass(h_hbm, e_hbm, ei_hbm, *refs):
    if with_deg:
      agg_hbm, deg_hbm = refs[0], refs[1]
      refs = refs[2:]
    else:
      agg_hbm = refs[0]
      refs = refs[1:]
    idx_v, g_buf, e_buf, z_buf, deg_t, accum, sem_g, sem_e = refs

    cid = lax.axis_index("c")
    sid = lax.axis_index("s")
    wid = cid * NSUB + sid
    base0 = sid * RPS

    @pl.loop(0, ZR)
    def _(r):
      for c in range(F // 16):
        z_buf[r, pl.ds(c * 16, 16)] = jnp.zeros((16,), jnp.float32)

    if with_deg:
      @pl.loop(0, NPAD // 16)
      def _(i):
        deg_t[pl.ds(i * 16, 16)] = jnp.zeros((16,), jnp.float32)

    _zero_accum(z_buf, accum, base0)
    plsc.subcore_barrier()

    @pl.loop(0, ITERS)
    def _(it):
      chunk = it * NW + wid

      @pl.when(chunk < NCHUNK)
      def _():
        base = pl.multiple_of(chunk * CH, CH)
        pltpu.sync_copy(ei_hbm.at[:, pl.ds(base, CH)], idx_v)
        cp_g = pltpu.async_copy(h_hbm.at[idx_v.at[0]], g_buf, sem_g)
        cp_e = pltpu.async_copy(e_hbm.at[chunk], e_buf, sem_e)
        cp_g.wait()
        cp_e.wait()

        @pl.loop(0, CH)
        def _(r):
          for c in range(F // 16):
            sl = (r, pl.ds(c * 16, 16))
            g_buf[sl] = jnp.maximum(g_buf[sl] + e_buf[sl], 0.0)

        pltpu.sync_copy(g_buf, accum.at[idx_v.at[1]], add=True)
        if with_deg:
          ones16 = jnp.ones((16,), jnp.float32)
          for c in range(CH // 16):
            d16 = idx_v[1, pl.ds(c * 16, 16)]
            plsc.addupdate_scatter(deg_t, [d16], ones16)

    plsc.subcore_barrier()
    pltpu.sync_copy(accum.at[pl.ds(base0, RPS)],
                    agg_hbm.at[cid, pl.ds(base0, RPS)])

    if with_deg:
      pltpu.sync_copy(deg_t, deg_hbm.at[wid])

  return gine_pass


_gine_pass_deg = _make_gine_pass(True)


NPAIR = 39
LAST_IT = 78


def _relu_add(g_buf, e_buf):
  @pl.loop(0, CH)
  def _(r):
    for c in range(F // 16):
      sl = (r, pl.ds(c * 16, 16))
      g_buf[sl] = jnp.maximum(g_buf[sl] + e_buf[sl], 0.0)


def _zero_accum_from(g_buf, accum, base0):
  @pl.loop(0, CH)
  def _(r):
    for c in range(F // 16):
      g_buf[r, pl.ds(c * 16, 16)] = jnp.zeros((16,), jnp.float32)
  for k in range(RPS // CH):
    pltpu.sync_copy(g_buf, accum.at[pl.ds(base0 + k * CH, CH)])
  rem = RPS % CH
  if rem:
    pltpu.sync_copy(g_buf.at[pl.ds(0, rem)],
                    accum.at[pl.ds(base0 + (RPS // CH) * CH, rem)])


@functools.partial(
    pl.kernel,
    out_type=jax.ShapeDtypeStruct((NCORE, NPAD, F), jnp.float32),
    mesh=_mesh,
    compiler_params=_sc_params,
    scratch_types=[
        pltpu.VMEM((2, CH), jnp.int32),
        pltpu.VMEM((2, CH), jnp.int32),
        pltpu.VMEM((CH, F), jnp.float32),
        pltpu.VMEM((CH, F), jnp.float32),
        pltpu.VMEM((CH, F), jnp.float32),
        pltpu.VMEM_SHARED((NPAD, F), jnp.float32),
        pltpu.SemaphoreType.DMA,
        pltpu.SemaphoreType.DMA,
        pltpu.SemaphoreType.DMA,
        pltpu.SemaphoreType.DMA,
        pltpu.SemaphoreType.DMA,
    ])
def _gine_pass(h_hbm, e_hbm, ei_hbm, agg_hbm, idx_a, idx_b, g_a, g_b, e_buf,
               accum, s_ia, s_ib, s_ga, s_gb, s_e):
  cid = lax.axis_index("c")
  sid = lax.axis_index("s")
  wid = cid * NSUB + sid
  base0 = sid * RPS

  _zero_accum_from(g_a, accum, base0)
  plsc.subcore_barrier()

  @pl.loop(0, NPAIR)
  def _(p):
    c0 = (2 * p) * NW + wid
    c1 = c0 + NW
    b0 = pl.multiple_of(c0 * CH, CH)
    b1 = pl.multiple_of(c1 * CH, CH)
    cpi0 = pltpu.async_copy(ei_hbm.at[:, pl.ds(b0, CH)], idx_a, s_ia)
    cpi1 = pltpu.async_copy(ei_hbm.at[:, pl.ds(b1, CH)], idx_b, s_ib)
    cpe0 = pltpu.async_copy(e_hbm.at[c0], e_buf, s_e)
    cpi0.wait()
    cpg0 = pltpu.async_copy(h_hbm.at[idx_a.at[0]], g_a, s_ga)
    cpi1.wait()
    cpg1 = pltpu.async_copy(h_hbm.at[idx_b.at[0]], g_b, s_gb)
    cpg0.wait()
    cpe0.wait()
    _relu_add(g_a, e_buf)
    pltpu.sync_copy(g_a, accum.at[idx_a.at[1]], add=True)
    cpe1 = pltpu.async_copy(e_hbm.at[c1], e_buf, s_e)
    cpg1.wait()
    cpe1.wait()
    _relu_add(g_b, e_buf)
    pltpu.sync_copy(g_b, accum.at[idx_b.at[1]], add=True)

  @pl.when(LAST_IT * NW + wid < NCHUNK)
  def _():
    c0 = LAST_IT * NW + wid
    b0 = pl.multiple_of(c0 * CH, CH)
    pltpu.sync_copy(ei_hbm.at[:, pl.ds(b0, CH)], idx_a)
    cpg = pltpu.async_copy(h_hbm.at[idx_a.at[0]], g_a, s_ga)
    cpe = pltpu.async_copy(e_hbm.at[c0], e_buf, s_e)
    cpg.wait()
    cpe.wait()
    _relu_add(g_a, e_buf)
    pltpu.sync_copy(g_a, accum.at[idx_a.at[1]], add=True)

  plsc.subcore_barrier()
  pltpu.sync_copy(accum.at[pl.ds(base0, RPS)],
                  agg_hbm.at[cid, pl.ds(base0, RPS)])


@functools.partial(
    pl.kernel,
    out_type=jax.ShapeDtypeStruct((NCORE, NPAD, F), jnp.float32),
    mesh=_mesh,
    compiler_params=_sc_params,
    scratch_types=[
        pltpu.VMEM((2, CH), jnp.int32),
        pltpu.VMEM((2, CH), jnp.int32),
        pltpu.VMEM((CH, F), jnp.float32),
        pltpu.VMEM((CH, F), jnp.float32),
        pltpu.VMEM_SHARED((NPAD, F), jnp.float32),
        pltpu.SemaphoreType.DMA,
        pltpu.SemaphoreType.DMA,
        pltpu.SemaphoreType.DMA,
        pltpu.SemaphoreType.DMA,
    ])
def _gcn_pass(y_hbm, ei_hbm, agg_hbm, idx_a, idx_b, g_a, g_b, accum,
              s_ia, s_ib, s_ga, s_gb):
  cid = lax.axis_index("c")
  sid = lax.axis_index("s")
  wid = cid * NSUB + sid
  base0 = sid * RPS

  _zero_accum_from(g_a, accum, base0)
  plsc.subcore_barrier()

  @pl.loop(0, NPAIR)
  def _(p):
    c0 = (2 * p) * NW + wid
    c1 = c0 + NW
    b0 = pl.multiple_of(c0 * CH, CH)
    b1 = pl.multiple_of(c1 * CH, CH)
    cpi0 = pltpu.async_copy(ei_hbm.at[:, pl.ds(b0, CH)], idx_a, s_ia)
    cpi1 = pltpu.async_copy(ei_hbm.at[:, pl.ds(b1, CH)], idx_b, s_ib)
    cpi0.wait()
    cpg0 = pltpu.async_copy(y_hbm.at[idx_a.at[0]], g_a, s_ga)
    cpi1.wait()
    cpg1 = pltpu.async_copy(y_hbm.at[idx_b.at[0]], g_b, s_gb)
    cpg0.wait()
    pltpu.sync_copy(g_a, accum.at[idx_a.at[1]], add=True)
    cpg1.wait()
    pltpu.sync_copy(g_b, accum.at[idx_b.at[1]], add=True)

  @pl.when(LAST_IT * NW + wid < NCHUNK)
  def _():
    c0 = LAST_IT * NW + wid
    b0 = pl.multiple_of(c0 * CH, CH)
    pltpu.sync_copy(ei_hbm.at[:, pl.ds(b0, CH)], idx_a)
    pltpu.async_copy(y_hbm.at[idx_a.at[0]], g_a, s_ga).wait()
    pltpu.sync_copy(g_a, accum.at[idx_a.at[1]], add=True)

  plsc.subcore_barrier()
  pltpu.sync_copy(accum.at[pl.ds(base0, RPS)],
                  agg_hbm.at[cid, pl.ds(base0, RPS)])


_BE = 4000
_BN = 2048


def _edge_mm_body(a_ref, w_ref, b_ref, o_ref):
  o_ref[...] = jnp.dot(a_ref[...], w_ref[...],
                       precision=lax.Precision.HIGHEST,
                       preferred_element_type=jnp.float32) + b_ref[...]


def _edge_mm(edge_attr, We, be):
  return pl.pallas_call(
      _edge_mm_body,
      grid=(E // _BE,),
      in_specs=[
          pl.BlockSpec((_BE, DE), lambda i: (i, 0)),
          pl.BlockSpec((DE, F), lambda i: (0, 0)),
          pl.BlockSpec((1, F), lambda i: (0, 0)),
      ],
      out_specs=pl.BlockSpec((_BE, F), lambda i: (i, 0)),
      out_shape=jax.ShapeDtypeStruct((E, F), jnp.float32),
  )(edge_attr, We, be.reshape(1, F))


def _sum_p(p_ref):
  return p_ref[0] + p_ref[1]


def _gine_node_body(x_ref, p_ref, w_ref, b_ref, o_ref):
  a = x_ref[...] + _sum_p(p_ref)
  h = jnp.dot(a, w_ref[...], precision=lax.Precision.HIGHEST,
              preferred_element_type=jnp.float32) + b_ref[...]
  o_ref[...] = jnp.maximum(h, 0.0)


def _gine_node(x, p, Wn, bn):
  return pl.pallas_call(
      _gine_node_body,
      grid=(pl.cdiv(N, _BN),),
      in_specs=[
          pl.BlockSpec((_BN, F), lambda i: (i, 0)),
          pl.BlockSpec((NCORE, _BN, F), lambda i: (0, i, 0)),
          pl.BlockSpec((F, F), lambda i: (0, 0)),
          pl.BlockSpec((1, F), lambda i: (0, 0)),
      ],
      out_specs=pl.BlockSpec((_BN, F), lambda i: (i, 0)),
      out_shape=jax.ShapeDtypeStruct((N, F), jnp.float32),
  )(x, p, Wn, bn.reshape(1, F))


def _gcn_prep_body(h_ref, p_ref, d_ref, wn_ref, bn_ref, wg_ref,
                   xw_ref, y_ref, dinv_ref):
  a = h_ref[...] + _sum_p(p_ref)
  h2 = jnp.maximum(
      jnp.dot(a, wn_ref[...], precision=lax.Precision.HIGHEST,
              preferred_element_type=jnp.float32) + bn_ref[...], 0.0)
  deg = jnp.sum(d_ref[...], axis=0) + 1.0
  dinv = lax.rsqrt(deg)[:, None]
  xw = jnp.dot(h2, wg_ref[...], precision=lax.Precision.HIGHEST,
               preferred_element_type=jnp.float32)
  xw_ref[...] = xw
  y_ref[...] = xw * dinv
  dinv_ref[...] = dinv


def _gcn_prep(h1, p, degp, W2n, b2n, Wg):
  return pl.pallas_call(
      _gcn_prep_body,
      grid=(pl.cdiv(N, _BN),),
      in_specs=[
          pl.BlockSpec((_BN, F), lambda i: (i, 0)),
          pl.BlockSpec((NCORE, _BN, F), lambda i: (0, i, 0)),
          pl.BlockSpec((NW, _BN), lambda i: (0, i)),
          pl.BlockSpec((F, F), lambda i: (0, 0)),
          pl.BlockSpec((1, F), lambda i: (0, 0)),
          pl.BlockSpec((F, F), lambda i: (0, 0)),
      ],
      out_specs=[
          pl.BlockSpec((_BN, F), lambda i: (i, 0)),
          pl.BlockSpec((_BN, F), lambda i: (i, 0)),
          pl.BlockSpec((_BN, 1), lambda i: (i, 0)),
      ],
      out_shape=[
          jax.ShapeDtypeStruct((N, F), jnp.float32),
          jax.ShapeDtypeStruct((N, F), jnp.float32),
          jax.ShapeDtypeStruct((N, 1), jnp.float32),
      ],
  )(h1, p, degp, W2n, b2n.reshape(1, F), Wg)


def _final_body(p_ref, xw_ref, dinv_ref, bg_ref, wo_ref, bo_ref, o_ref):
  dv = dinv_ref[...]
  h3 = jnp.maximum(dv * _sum_p(p_ref) + dv * dv * xw_ref[...]
                   + bg_ref[...], 0.0)
  logits = jnp.dot(h3, wo_ref[...], precision=lax.Precision.HIGHEST,
                   preferred_element_type=jnp.float32) + bo_ref[...]
  m = jnp.max(logits, axis=1, keepdims=True)
  lse = m + jnp.log(jnp.sum(jnp.exp(logits - m), axis=1, keepdims=True))
  o_ref[...] = (logits - lse)[:, :C]


def _final(p, xw, dinv, bg, Wo_pad, bo_pad):
  return pl.pallas_call(
      _final_body,
      grid=(pl.cdiv(N, _BN),),
      in_specs=[
          pl.BlockSpec((NCORE, _BN, F), lambda i: (0, i, 0)),
          pl.BlockSpec((_BN, F), lambda i: (i, 0)),
          pl.BlockSpec((_BN, 1), lambda i: (i, 0)),
          pl.BlockSpec((1, F), lambda i: (0, 0)),
          pl.BlockSpec((F, F), lambda i: (0, 0)),
          pl.BlockSpec((1, F), lambda i: (0, 0)),
      ],
      out_specs=pl.BlockSpec((_BN, C), lambda i: (i, 0)),
      out_shape=jax.ShapeDtypeStruct((N, C), jnp.float32),
  )(p, xw, dinv, bg.reshape(1, F), Wo_pad, bo_pad.reshape(1, F))


def kernel(x, edge_index, edge_attr, W1n, b1n, W1e, b1e, W2n, b2n, W2e, b2e,
           Wg, bg, Wo, bo):
  e1 = _edge_mm(edge_attr, W1e, b1e).reshape(NCHUNK, CH, F)
  e2 = _edge_mm(edge_attr, W2e, b2e).reshape(NCHUNK, CH, F)

  p1, degp = _gine_pass_deg(x, e1, edge_index)
  h1 = _gine_node(x, p1, W1n, b1n)

  p2 = _gine_pass(h1, e2, edge_index)
  xw, y, dinv = _gcn_prep(h1, p2, degp, W2n, b2n, Wg)

  p3 = _gcn_pass(y, edge_index)

  Wo_pad = jnp.pad(Wo, ((0, 0), (0, F - C)))
  bo_pad = jnp.pad(bo, (0, F - C), constant_values=-1e30)
  return _final(p3, xw, dinv, bg, Wo_pad, bo_pad)
